# baseline (speedup 1.0000x reference)
import jax
import jax.numpy as jnp
import numpy as np
from jax import lax
from jax.experimental import pallas as pl
from jax.experimental.pallas import tpu as pltpu
from jax.experimental.pallas import tpu_sc as plsc

N_NODES = 10000
N_INC = 320000
D = 128
K = 512
L = 2
CC = 0.25
BN_SCALE = float(1.0 / np.sqrt(1.0 + 1e-5))

R_A = 2000
R_B = 1000

NW = 32
PAD_TOT = 10240
SLAB = PAD_TOT // 16
PERW = N_INC // NW
NFULL = PERW // 128
TAIL = PERW - NFULL * 128

_MESH = dict(core_axis_name="c", subcore_axis_name="s", num_cores=2,
             num_subcores=16)


def _pass_body(table_h, g_h, d_h, out_h,
               acc, rows, gb, db, gbt, dbt, rowst, sem):
    c = lax.axis_index("c")
    s = lax.axis_index("s")
    wid = s * 2 + c

    def zero_body(j, car):
        for cc in range(8):
            rows[j, pl.ds(cc * 16, 16)] = jnp.zeros((16,), jnp.float32)
        return car
    lax.fori_loop(0, 128, zero_body, 0)
    for j in range(SLAB // 128):
        pltpu.sync_copy(rows, acc.at[pl.ds(s * SLAB + j * 128, 128)])
    plsc.subcore_barrier()

    i0 = wid * PERW

    def blk_body(k, car):
        off = pl.multiple_of(i0 + k * 128, 8)
        pltpu.sync_copy(g_h.at[pl.ds(off, 128)], gb.at[0])
        pltpu.sync_copy(d_h.at[pl.ds(off, 128)], db.at[0])
        pltpu.async_copy(table_h.at[gb.at[0]], rows, sem).wait()
        pltpu.sync_copy(rows, acc.at[db.at[0]], add=True)
        return car
    lax.fori_loop(0, NFULL, blk_body, 0)

    offt = pl.multiple_of(i0 + NFULL * 128, 8)
    pltpu.sync_copy(g_h.at[pl.ds(offt, TAIL)], gbt.at[0])
    pltpu.sync_copy(d_h.at[pl.ds(offt, TAIL)], dbt.at[0])
    pltpu.async_copy(table_h.at[gbt.at[0]], rowst, sem).wait()
    pltpu.sync_copy(rowst, acc.at[dbt.at[0]], add=True)

    plsc.subcore_barrier()
    pltpu.sync_copy(acc.at[pl.ds(s * SLAB, SLAB)],
                    out_h.at[c, pl.ds(s * SLAB, SLAB)])


def _sc_pass(table, gidx, didx):
    f32 = jnp.float32
    f = pl.kernel(
        _pass_body,
        out_type=[
            jax.ShapeDtypeStruct((2, PAD_TOT, D), f32),
        ],
        mesh=plsc.VectorSubcoreMesh(**_MESH),
        scratch_types=[
            pltpu.VMEM_SHARED((PAD_TOT, D), f32),
            pltpu.VMEM((128, D), f32),
            pltpu.VMEM((1, 128), jnp.int32),
            pltpu.VMEM((1, 128), jnp.int32),
            pltpu.VMEM((1, TAIL), jnp.int32),
            pltpu.VMEM((1, TAIL), jnp.int32),
            pltpu.VMEM((TAIL, D), f32),
            pltpu.SemaphoreType.DMA,
        ],
    )
    return f(table, gidx, didx)[0]


def _conv_in_body(x_ref, g_ref, b_ref, w_ref, o_ref):
    h = g_ref[...] * (x_ref[...] * BN_SCALE) + b_ref[...]
    o_ref[...] = jnp.dot(h, w_ref[...])


def _conv_in(X, g, b, W):
    return pl.pallas_call(
        _conv_in_body,
        grid=(N_NODES // R_A,),
        in_specs=[
            pl.BlockSpec((R_A, D), lambda i: (i, 0)),
            pl.BlockSpec((1, D), lambda i: (0, 0)),
            pl.BlockSpec((1, D), lambda i: (0, 0)),
            pl.BlockSpec((D, D), lambda i: (0, 0)),
        ],
        out_specs=pl.BlockSpec((R_A, D), lambda i: (i, 0)),
        out_shape=jax.ShapeDtypeStruct((N_NODES, D), jnp.float32),
    )(X, g.reshape(1, D), b.reshape(1, D), W)


def _scale_body(m_ref, h_ref, o_ref):
    msum = m_ref[0] + m_ref[1]
    cnt = (h_ref[0] + h_ref[1])[:, 0:1]
    inv = jnp.where(cnt > 0, 1.0 / cnt, 0.0)
    o_ref[...] = msum * inv


def _scale(parts, hist):
    blk = 2048
    return pl.pallas_call(
        _scale_body,
        grid=(PAD_TOT // blk,),
        in_specs=[
            pl.BlockSpec((2, blk, D), lambda i: (0, i, 0)),
            pl.BlockSpec((2, blk, D), lambda i: (0, i, 0)),
        ],
        out_specs=pl.BlockSpec((blk, D), lambda i: (i, 0)),
        out_shape=jax.ShapeDtypeStruct((PAD_TOT, D), jnp.float32),
    )(parts, hist)


def _vq_body(conv_ref, hist_ref, x_ref, gum_ref, cb_ref, cbias_ref, gg_ref,
             gb_ref, gw_ref, gbias_ref, xo_ref, loss_ref, perp_ref,
             ll_acc, cnt_acc):
    i = pl.program_id(0)
    nblk = pl.num_programs(0)

    @pl.when(i == 0)
    def _init():
        ll_acc[0, 0] = jnp.float32(0.0)
        cnt_acc[...] = jnp.zeros_like(cnt_acc)

    dcol = (hist_ref[0] + hist_ref[1])[:, 0:1]
    dinv = jnp.where(dcol > 0, 1.0 / dcol, 0.0)
    conv = (conv_ref[0] + conv_ref[1]) * dinv
    h = jnp.maximum(conv + cbias_ref[...], 0.0)
    gx = gg_ref[...] * (x_ref[...] * BN_SCALE) + gb_ref[...]
    glogit = jnp.sum(gx * gw_ref[...], axis=1, keepdims=True) + gbias_ref[...]
    gate = jax.nn.sigmoid(glogit)
    msg = h * gate

    cb = cb_ref[...]
    m2 = jnp.sum(msg * msg, axis=1, keepdims=True)
    c2 = jnp.sum(cb * cb, axis=1)
    gmat = lax.dot_general(msg, cb, (((1,), (1,)), ((), ())))
    dist = m2 + c2[None, :] - 2.0 * gmat

    sneg = -dist
    smax = jnp.max(sneg, axis=1, keepdims=True)
    e = jnp.exp(sneg - smax)
    z = jnp.sum(e, axis=1, keepdims=True)
    soft = e / z
    ll_rows = jnp.sum(soft * jnp.log(jnp.maximum(soft, 1e-8)), axis=1)
    ll_acc[0, 0] += jnp.sum(ll_rows)

    score = sneg + gum_ref[...]
    mx = jnp.max(score, axis=1, keepdims=True)
    iota = lax.broadcasted_iota(jnp.int32, score.shape, 1)
    cand = jnp.where(score == mx, iota, K)
    idx = jnp.min(cand, axis=1, keepdims=True)
    enc = (iota == idx).astype(jnp.float32)

    quant = jnp.dot(enc, cb)
    cnt_acc[...] += jnp.sum(enc, axis=0, keepdims=True)
    xo_ref[...] = x_ref[...] + quant

    @pl.when(i == nblk - 1)
    def _fin():
        loss_ref[...] = jnp.full((1, 1), CC * (ll_acc[0, 0] / N_NODES),
                                 jnp.float32)
        avg = cnt_acc[...] / N_NODES
        perp_ref[...] = jnp.full(
            (1, 1), jnp.exp(-jnp.sum(avg * jnp.log(avg + 1e-10))), jnp.float32)


def _vq_stage(conv_parts, hist, X, gum, cb, conv_b, gg, gb, gw, gbias):
    xo, loss, perp = pl.pallas_call(
        _vq_body,
        grid=(N_NODES // R_B,),
        in_specs=[
            pl.BlockSpec((2, R_B, D), lambda i: (0, i, 0)),
            pl.BlockSpec((2, R_B, D), lambda i: (0, i, 0)),
            pl.BlockSpec((R_B, D), lambda i: (i, 0)),
            pl.BlockSpec((R_B, K), lambda i: (i, 0)),
            pl.BlockSpec((K, D), lambda i: (0, 0)),
            pl.BlockSpec((1, D), lambda i: (0, 0)),
            pl.BlockSpec((1, D), lambda i: (0, 0)),
            pl.BlockSpec((1, D), lambda i: (0, 0)),
            pl.BlockSpec((1, D), lambda i: (0, 0)),
            pl.BlockSpec((1, 1), lambda i: (0, 0)),
        ],
        out_specs=[
            pl.BlockSpec((R_B, D), lambda i: (i, 0)),
            pl.BlockSpec((1, 1), lambda i: (0, 0)),
            pl.BlockSpec((1, 1), lambda i: (0, 0)),
        ],
        out_shape=[
            jax.ShapeDtypeStruct((N_NODES, D), jnp.float32),
            jax.ShapeDtypeStruct((1, 1), jnp.float32),
            jax.ShapeDtypeStruct((1, 1), jnp.float32),
        ],
        scratch_shapes=[
            pltpu.SMEM((1, 1), jnp.float32),
            pltpu.VMEM((1, K), jnp.float32),
        ],
    )(conv_parts, hist, X, gum, cb, conv_b.reshape(1, D), gg.reshape(1, D),
      gb.reshape(1, D), gw.reshape(1, D), gbias.reshape(1, 1))
    return xo, loss[0, 0], perp[0, 0]


def kernel(X, H, params, codebooks):
    src, edge = H[0], H[1]
    ones_table = jnp.ones((8, D), jnp.float32)
    zidx = jnp.zeros((N_INC,), jnp.int32)
    hsrc = _sc_pass(ones_table, zidx, src)
    hedge = _sc_pass(ones_table, zidx, edge)
    base = jax.random.key(42)
    loss_latents = jnp.float32(0.0)
    perp = jnp.float32(0.0)
    for i in range(L):
        p = params[i]
        gum = jax.random.gumbel(jax.random.fold_in(base, i), (N_NODES, K),
                                dtype=jnp.float32)
        xW = _conv_in(X, p['bn_g'], p['bn_b'], p['conv_W'])
        m_parts = _sc_pass(xW, src, edge)
        m = _scale(m_parts, hedge)
        out_parts = _sc_pass(m, edge, src)
        X, loss, perp = _vq_stage(out_parts, hsrc, X, gum, codebooks[i],
                                  p['conv_b'], p['gbn_g'], p['gbn_b'],
                                  p['gate_W'][:, 0], p['gate_b'])
        loss_latents = loss_latents + loss
    return X, loss_latents, perp


# speedup vs baseline: 1.0127x; 1.0127x over previous
import jax
import jax.numpy as jnp
import numpy as np
from jax import lax
from jax.experimental import pallas as pl
from jax.experimental.pallas import tpu as pltpu
from jax.experimental.pallas import tpu_sc as plsc

N_NODES = 10000
N_INC = 320000
D = 128
K = 512
L = 2
CC = 0.25
BN_SCALE = float(1.0 / np.sqrt(1.0 + 1e-5))

R_A = 2000
R_B = 1000

NW = 32
PAD_TOT = 10240
SLAB = PAD_TOT // 16
PERW = N_INC // NW
NFULL = PERW // 128
TAIL = PERW - NFULL * 128

_MESH = dict(core_axis_name="c", subcore_axis_name="s", num_cores=2,
             num_subcores=16)


def _pass_body(table_h, g_h, d_h, out_h,
               acc, rows, gbf, db, semi, semg, sems):
    c = lax.axis_index("c")
    s = lax.axis_index("s")
    wid = s * 2 + c

    def zero_body(j, car):
        for cc in range(8):
            rows[j, pl.ds(cc * 16, 16)] = jnp.zeros((16,), jnp.float32)
        return car
    lax.fori_loop(0, 128, zero_body, 0)
    for j in range(SLAB // 128):
        pltpu.sync_copy(rows.at[pl.ds(0, 128)],
                        acc.at[pl.ds(s * SLAB + j * 128, 128)])
    plsc.subcore_barrier()

    i0 = wid * PERW

    def chunk(off, nb):
        gcp = pltpu.make_async_copy(g_h.at[pl.ds(off, nb * 128)],
                                    gbf.at[pl.ds(0, nb * 128)], semi)
        gcp.start()
        dcps = []
        for b in range(nb):
            cp = pltpu.make_async_copy(
                d_h.at[pl.ds(off + b * 128, 128)], db.at[b], semi)
            cp.start()
            dcps.append(cp)
        gcp.wait()
        for cp in dcps:
            cp.wait()
        gcps = []
        for b in range(nb):
            cp = pltpu.make_async_copy(
                table_h.at[gbf.at[pl.ds(b * 128, 128)]],
                rows.at[pl.ds(b * 128, 128)], semg)
            cp.start()
            gcps.append(cp)
        for cp in gcps:
            cp.wait()
        for b in range(nb):
            pltpu.async_copy(rows.at[pl.ds(b * 128, 128)],
                             acc.at[db.at[b]], sems, add=True).wait()

    def blk_body(k, car):
        chunk(pl.multiple_of(i0 + k * 256, 8), 2)
        return car
    lax.fori_loop(0, NFULL // 2, blk_body, 0)

    offt = pl.multiple_of(i0 + NFULL * 128, 8)
    pltpu.sync_copy(g_h.at[pl.ds(offt, TAIL)], gbf.at[pl.ds(0, TAIL)])
    pltpu.sync_copy(d_h.at[pl.ds(offt, TAIL)], db.at[0, pl.ds(0, TAIL)])
    pltpu.async_copy(table_h.at[gbf.at[pl.ds(0, TAIL)]],
                     rows.at[pl.ds(0, TAIL)], semg).wait()
    pltpu.async_copy(rows.at[pl.ds(0, TAIL)],
                     acc.at[db.at[0, pl.ds(0, TAIL)]], sems, add=True).wait()

    plsc.subcore_barrier()
    pltpu.sync_copy(acc.at[pl.ds(s * SLAB, SLAB)],
                    out_h.at[c, pl.ds(s * SLAB, SLAB)])


def _sc_pass(table, gidx, didx):
    f32 = jnp.float32
    f = pl.kernel(
        _pass_body,
        out_type=[
            jax.ShapeDtypeStruct((2, PAD_TOT, D), f32),
        ],
        mesh=plsc.VectorSubcoreMesh(**_MESH),
        scratch_types=[
            pltpu.VMEM_SHARED((PAD_TOT, D), f32),
            pltpu.VMEM((256, D), f32),
            pltpu.VMEM((256,), jnp.int32),
            pltpu.VMEM((2, 128), jnp.int32),
            pltpu.SemaphoreType.DMA,
            pltpu.SemaphoreType.DMA,
            pltpu.SemaphoreType.DMA,
        ],
    )
    return f(table, gidx, didx)[0]


def _conv_in_body(x_ref, g_ref, b_ref, w_ref, o_ref):
    h = g_ref[...] * (x_ref[...] * BN_SCALE) + b_ref[...]
    o_ref[...] = jnp.dot(h, w_ref[...])


def _conv_in(X, g, b, W):
    return pl.pallas_call(
        _conv_in_body,
        grid=(N_NODES // R_A,),
        in_specs=[
            pl.BlockSpec((R_A, D), lambda i: (i, 0)),
            pl.BlockSpec((1, D), lambda i: (0, 0)),
            pl.BlockSpec((1, D), lambda i: (0, 0)),
            pl.BlockSpec((D, D), lambda i: (0, 0)),
        ],
        out_specs=pl.BlockSpec((R_A, D), lambda i: (i, 0)),
        out_shape=jax.ShapeDtypeStruct((N_NODES, D), jnp.float32),
    )(X, g.reshape(1, D), b.reshape(1, D), W)


def _scale_body(m_ref, h_ref, o_ref):
    msum = m_ref[0] + m_ref[1]
    cnt = (h_ref[0] + h_ref[1])[:, 0:1]
    inv = jnp.where(cnt > 0, 1.0 / cnt, 0.0)
    o_ref[...] = msum * inv


def _scale(parts, hist):
    blk = 2048
    return pl.pallas_call(
        _scale_body,
        grid=(PAD_TOT // blk,),
        in_specs=[
            pl.BlockSpec((2, blk, D), lambda i: (0, i, 0)),
            pl.BlockSpec((2, blk, D), lambda i: (0, i, 0)),
        ],
        out_specs=pl.BlockSpec((blk, D), lambda i: (i, 0)),
        out_shape=jax.ShapeDtypeStruct((PAD_TOT, D), jnp.float32),
    )(parts, hist)


def _vq_body(conv_ref, hist_ref, x_ref, gum_ref, cb_ref, cbias_ref, gg_ref,
             gb_ref, gw_ref, gbias_ref, xo_ref, loss_ref, perp_ref,
             ll_acc, cnt_acc):
    i = pl.program_id(0)
    nblk = pl.num_programs(0)

    @pl.when(i == 0)
    def _init():
        ll_acc[0, 0] = jnp.float32(0.0)
        cnt_acc[...] = jnp.zeros_like(cnt_acc)

    dcol = (hist_ref[0] + hist_ref[1])[:, 0:1]
    dinv = jnp.where(dcol > 0, 1.0 / dcol, 0.0)
    conv = (conv_ref[0] + conv_ref[1]) * dinv
    h = jnp.maximum(conv + cbias_ref[...], 0.0)
    gx = gg_ref[...] * (x_ref[...] * BN_SCALE) + gb_ref[...]
    glogit = jnp.sum(gx * gw_ref[...], axis=1, keepdims=True) + gbias_ref[...]
    gate = jax.nn.sigmoid(glogit)
    msg = h * gate

    cb = cb_ref[...]
    m2 = jnp.sum(msg * msg, axis=1, keepdims=True)
    c2 = jnp.sum(cb * cb, axis=1)
    gmat = lax.dot_general(msg, cb, (((1,), (1,)), ((), ())))
    dist = m2 + c2[None, :] - 2.0 * gmat

    sneg = -dist
    smax = jnp.max(sneg, axis=1, keepdims=True)
    e = jnp.exp(sneg - smax)
    z = jnp.sum(e, axis=1, keepdims=True)
    soft = e / z
    ll_rows = jnp.sum(soft * jnp.log(jnp.maximum(soft, 1e-8)), axis=1)
    ll_acc[0, 0] += jnp.sum(ll_rows)

    score = sneg + gum_ref[...]
    mx = jnp.max(score, axis=1, keepdims=True)
    iota = lax.broadcasted_iota(jnp.int32, score.shape, 1)
    cand = jnp.where(score == mx, iota, K)
    idx = jnp.min(cand, axis=1, keepdims=True)
    enc = (iota == idx).astype(jnp.float32)

    quant = jnp.dot(enc, cb)
    cnt_acc[...] += jnp.sum(enc, axis=0, keepdims=True)
    xo_ref[...] = x_ref[...] + quant

    @pl.when(i == nblk - 1)
    def _fin():
        loss_ref[...] = jnp.full((1, 1), CC * (ll_acc[0, 0] / N_NODES),
                                 jnp.float32)
        avg = cnt_acc[...] / N_NODES
        perp_ref[...] = jnp.full(
            (1, 1), jnp.exp(-jnp.sum(avg * jnp.log(avg + 1e-10))), jnp.float32)


def _vq_stage(conv_parts, hist, X, gum, cb, conv_b, gg, gb, gw, gbias):
    xo, loss, perp = pl.pallas_call(
        _vq_body,
        grid=(N_NODES // R_B,),
        in_specs=[
            pl.BlockSpec((2, R_B, D), lambda i: (0, i, 0)),
            pl.BlockSpec((2, R_B, D), lambda i: (0, i, 0)),
            pl.BlockSpec((R_B, D), lambda i: (i, 0)),
            pl.BlockSpec((R_B, K), lambda i: (i, 0)),
            pl.BlockSpec((K, D), lambda i: (0, 0)),
            pl.BlockSpec((1, D), lambda i: (0, 0)),
            pl.BlockSpec((1, D), lambda i: (0, 0)),
            pl.BlockSpec((1, D), lambda i: (0, 0)),
            pl.BlockSpec((1, D), lambda i: (0, 0)),
            pl.BlockSpec((1, 1), lambda i: (0, 0)),
        ],
        out_specs=[
            pl.BlockSpec((R_B, D), lambda i: (i, 0)),
            pl.BlockSpec((1, 1), lambda i: (0, 0)),
            pl.BlockSpec((1, 1), lambda i: (0, 0)),
        ],
        out_shape=[
            jax.ShapeDtypeStruct((N_NODES, D), jnp.float32),
            jax.ShapeDtypeStruct((1, 1), jnp.float32),
            jax.ShapeDtypeStruct((1, 1), jnp.float32),
        ],
        scratch_shapes=[
            pltpu.SMEM((1, 1), jnp.float32),
            pltpu.VMEM((1, K), jnp.float32),
        ],
    )(conv_parts, hist, X, gum, cb, conv_b.reshape(1, D), gg.reshape(1, D),
      gb.reshape(1, D), gw.reshape(1, D), gbias.reshape(1, 1))
    return xo, loss[0, 0], perp[0, 0]


def kernel(X, H, params, codebooks):
    src, edge = H[0], H[1]
    ones_table = jnp.ones((8, D), jnp.float32)
    zidx = jnp.zeros((N_INC,), jnp.int32)
    hsrc = _sc_pass(ones_table, zidx, src)
    hedge = _sc_pass(ones_table, zidx, edge)
    base = jax.random.key(42)
    loss_latents = jnp.float32(0.0)
    perp = jnp.float32(0.0)
    for i in range(L):
        p = params[i]
        gum = jax.random.gumbel(jax.random.fold_in(base, i), (N_NODES, K),
                                dtype=jnp.float32)
        xW = _conv_in(X, p['bn_g'], p['bn_b'], p['conv_W'])
        m_parts = _sc_pass(xW, src, edge)
        m = _scale(m_parts, hedge)
        out_parts = _sc_pass(m, edge, src)
        X, loss, perp = _vq_stage(out_parts, hsrc, X, gum, codebooks[i],
                                  p['conv_b'], p['gbn_g'], p['gbn_b'],
                                  p['gate_W'][:, 0], p['gate_b'])
        loss_latents = loss_latents + loss
    return X, loss_latents, perp


# speedup vs baseline: 22.3344x; 22.0549x over previous
import jax
import jax.numpy as jnp
import numpy as np
from jax import lax
from jax.experimental import pallas as pl
from jax.experimental.pallas import tpu as pltpu
from jax.experimental.pallas import tpu_sc as plsc

N_NODES = 10000
N_INC = 320000
D = 128
K = 512
L = 2
CC = 0.25
BN_SCALE = float(1.0 / np.sqrt(1.0 + 1e-5))

R_A = 2000
R_B = 1000

NW = 32
PAD_TOT = 10240
SLAB = PAD_TOT // 16
PERW = N_INC // NW
NFULL = PERW // 128
TAIL = PERW - NFULL * 128

_MESH = dict(core_axis_name="c", subcore_axis_name="s", num_cores=2,
             num_subcores=16)


def _pass_body(table_h, g_h, d_h, out_h,
               acc, rows, gbf, db, semi, semg, sems):
    c = lax.axis_index("c")
    s = lax.axis_index("s")
    wid = s * 2 + c

    def zero_body(j, car):
        for cc in range(8):
            rows[j, pl.ds(cc * 16, 16)] = jnp.zeros((16,), jnp.float32)
        return car
    lax.fori_loop(0, 128, zero_body, 0)
    for j in range(SLAB // 128):
        pltpu.sync_copy(rows.at[pl.ds(0, 128)],
                        acc.at[pl.ds(s * SLAB + j * 128, 128)])
    plsc.subcore_barrier()

    i0 = wid * PERW

    def chunk(off, nb):
        gcp = pltpu.make_async_copy(g_h.at[pl.ds(off, nb * 128)],
                                    gbf.at[pl.ds(0, nb * 128)], semi)
        gcp.start()
        dcps = []
        for b in range(nb):
            cp = pltpu.make_async_copy(
                d_h.at[pl.ds(off + b * 128, 128)], db.at[b], semi)
            cp.start()
            dcps.append(cp)
        gcp.wait()
        for cp in dcps:
            cp.wait()
        gcps = []
        for b in range(nb):
            cp = pltpu.make_async_copy(
                table_h.at[gbf.at[pl.ds(b * 128, 128)]],
                rows.at[pl.ds(b * 128, 128)], semg)
            cp.start()
            gcps.append(cp)
        for cp in gcps:
            cp.wait()
        for b in range(nb):
            pltpu.async_copy(rows.at[pl.ds(b * 128, 128)],
                             acc.at[db.at[b]], sems, add=True).wait()

    def blk_body(k, car):
        chunk(pl.multiple_of(i0 + k * 256, 8), 2)
        return car
    lax.fori_loop(0, NFULL // 2, blk_body, 0)

    offt = pl.multiple_of(i0 + NFULL * 128, 8)
    pltpu.sync_copy(g_h.at[pl.ds(offt, TAIL)], gbf.at[pl.ds(0, TAIL)])
    pltpu.sync_copy(d_h.at[pl.ds(offt, TAIL)], db.at[0, pl.ds(0, TAIL)])
    pltpu.async_copy(table_h.at[gbf.at[pl.ds(0, TAIL)]],
                     rows.at[pl.ds(0, TAIL)], semg).wait()
    pltpu.async_copy(rows.at[pl.ds(0, TAIL)],
                     acc.at[db.at[0, pl.ds(0, TAIL)]], sems, add=True).wait()

    plsc.subcore_barrier()
    pltpu.sync_copy(acc.at[pl.ds(s * SLAB, SLAB)],
                    out_h.at[c, pl.ds(s * SLAB, SLAB)])


def _sc_pass(table, gidx, didx):
    f32 = jnp.float32
    f = pl.kernel(
        _pass_body,
        out_type=[
            jax.ShapeDtypeStruct((2, PAD_TOT, D), f32),
        ],
        mesh=plsc.VectorSubcoreMesh(**_MESH),
        scratch_types=[
            pltpu.VMEM_SHARED((PAD_TOT, D), f32),
            pltpu.VMEM((256, D), f32),
            pltpu.VMEM((256,), jnp.int32),
            pltpu.VMEM((2, 128), jnp.int32),
            pltpu.SemaphoreType.DMA,
            pltpu.SemaphoreType.DMA,
            pltpu.SemaphoreType.DMA,
        ],
    )
    return f(table, gidx, didx)[0]


def _conv_in_body(x_ref, g_ref, b_ref, w_ref, o_ref):
    h = g_ref[...] * (x_ref[...] * BN_SCALE) + b_ref[...]
    o_ref[...] = jnp.dot(h, w_ref[...])


def _conv_in(X, g, b, W):
    return pl.pallas_call(
        _conv_in_body,
        grid=(N_NODES // R_A,),
        in_specs=[
            pl.BlockSpec((R_A, D), lambda i: (i, 0)),
            pl.BlockSpec((1, D), lambda i: (0, 0)),
            pl.BlockSpec((1, D), lambda i: (0, 0)),
            pl.BlockSpec((D, D), lambda i: (0, 0)),
        ],
        out_specs=pl.BlockSpec((R_A, D), lambda i: (i, 0)),
        out_shape=jax.ShapeDtypeStruct((N_NODES, D), jnp.float32),
    )(X, g.reshape(1, D), b.reshape(1, D), W)


def _scale_body(m_ref, h_ref, o_ref):
    msum = m_ref[0] + m_ref[1]
    cnt = (h_ref[0] + h_ref[1])[:, 0:1]
    inv = jnp.where(cnt > 0, 1.0 / cnt, 0.0)
    o_ref[...] = msum * inv


def _scale(parts, hist):
    blk = 2048
    return pl.pallas_call(
        _scale_body,
        grid=(PAD_TOT // blk,),
        in_specs=[
            pl.BlockSpec((2, blk, D), lambda i: (0, i, 0)),
            pl.BlockSpec((2, blk, D), lambda i: (0, i, 0)),
        ],
        out_specs=pl.BlockSpec((blk, D), lambda i: (i, 0)),
        out_shape=jax.ShapeDtypeStruct((PAD_TOT, D), jnp.float32),
    )(parts, hist)


def _vq_body(conv_ref, hist_ref, x_ref, gum_ref, cb_ref, cbias_ref, gg_ref,
             gb_ref, gw_ref, gbias_ref, xo_ref, loss_ref, perp_ref,
             ll_acc, cnt_acc):
    i = pl.program_id(0)
    nblk = pl.num_programs(0)

    @pl.when(i == 0)
    def _init():
        ll_acc[0, 0] = jnp.float32(0.0)
        cnt_acc[...] = jnp.zeros_like(cnt_acc)

    dcol = (hist_ref[0] + hist_ref[1])[:, 0:1]
    dinv = jnp.where(dcol > 0, 1.0 / dcol, 0.0)
    conv = (conv_ref[0] + conv_ref[1]) * dinv
    h = jnp.maximum(conv + cbias_ref[...], 0.0)
    gx = gg_ref[...] * (x_ref[...] * BN_SCALE) + gb_ref[...]
    glogit = jnp.sum(gx * gw_ref[...], axis=1, keepdims=True) + gbias_ref[...]
    gate = jax.nn.sigmoid(glogit)
    msg = h * gate

    cb = cb_ref[...]
    m2 = jnp.sum(msg * msg, axis=1, keepdims=True)
    c2 = jnp.sum(cb * cb, axis=1)
    gmat = lax.dot_general(msg, cb, (((1,), (1,)), ((), ())))
    dist = m2 + c2[None, :] - 2.0 * gmat

    sneg = -dist
    smax = jnp.max(sneg, axis=1, keepdims=True)
    e = jnp.exp(sneg - smax)
    z = jnp.sum(e, axis=1, keepdims=True)
    soft = e / z
    ll_rows = jnp.sum(soft * jnp.log(jnp.maximum(soft, 1e-8)), axis=1)
    ll_acc[0, 0] += jnp.sum(ll_rows)

    score = sneg + gum_ref[...]
    mx = jnp.max(score, axis=1, keepdims=True)
    iota = lax.broadcasted_iota(jnp.int32, score.shape, 1)
    cand = jnp.where(score == mx, iota, K)
    idx = jnp.min(cand, axis=1, keepdims=True)
    enc = (iota == idx).astype(jnp.float32)

    quant = jnp.dot(enc, cb)
    cnt_acc[...] += jnp.sum(enc, axis=0, keepdims=True)
    xo_ref[...] = x_ref[...] + quant

    @pl.when(i == nblk - 1)
    def _fin():
        loss_ref[...] = jnp.full((1, 1), CC * (ll_acc[0, 0] / N_NODES),
                                 jnp.float32)
        avg = cnt_acc[...] / N_NODES
        perp_ref[...] = jnp.full(
            (1, 1), jnp.exp(-jnp.sum(avg * jnp.log(avg + 1e-10))), jnp.float32)


def _vq_stage(conv_parts, hist, X, gum, cb, conv_b, gg, gb, gw, gbias):
    xo, loss, perp = pl.pallas_call(
        _vq_body,
        grid=(N_NODES // R_B,),
        in_specs=[
            pl.BlockSpec((2, R_B, D), lambda i: (0, i, 0)),
            pl.BlockSpec((2, R_B, D), lambda i: (0, i, 0)),
            pl.BlockSpec((R_B, D), lambda i: (i, 0)),
            pl.BlockSpec((R_B, K), lambda i: (i, 0)),
            pl.BlockSpec((K, D), lambda i: (0, 0)),
            pl.BlockSpec((1, D), lambda i: (0, 0)),
            pl.BlockSpec((1, D), lambda i: (0, 0)),
            pl.BlockSpec((1, D), lambda i: (0, 0)),
            pl.BlockSpec((1, D), lambda i: (0, 0)),
            pl.BlockSpec((1, 1), lambda i: (0, 0)),
        ],
        out_specs=[
            pl.BlockSpec((R_B, D), lambda i: (i, 0)),
            pl.BlockSpec((1, 1), lambda i: (0, 0)),
            pl.BlockSpec((1, 1), lambda i: (0, 0)),
        ],
        out_shape=[
            jax.ShapeDtypeStruct((N_NODES, D), jnp.float32),
            jax.ShapeDtypeStruct((1, 1), jnp.float32),
            jax.ShapeDtypeStruct((1, 1), jnp.float32),
        ],
        scratch_shapes=[
            pltpu.SMEM((1, 1), jnp.float32),
            pltpu.VMEM((1, K), jnp.float32),
        ],
    )(conv_parts, hist, X, gum, cb, conv_b.reshape(1, D), gg.reshape(1, D),
      gb.reshape(1, D), gw.reshape(1, D), gbias.reshape(1, 1))
    return xo, loss[0, 0], perp[0, 0]


def kernel(X, H, params, codebooks):
    src, edge = H[0], H[1]
    ones_table = jnp.ones((N_NODES, D), jnp.float32)
    hsrc = _sc_pass(ones_table, src, src)
    hedge = _sc_pass(ones_table, edge, edge)
    base = jax.random.key(42)
    loss_latents = jnp.float32(0.0)
    perp = jnp.float32(0.0)
    for i in range(L):
        p = params[i]
        gum = jax.random.gumbel(jax.random.fold_in(base, i), (N_NODES, K),
                                dtype=jnp.float32)
        xW = _conv_in(X, p['bn_g'], p['bn_b'], p['conv_W'])
        m_parts = _sc_pass(xW, src, edge)
        m = _scale(m_parts, hedge)
        out_parts = _sc_pass(m, edge, src)
        X, loss, perp = _vq_stage(out_parts, hsrc, X, gum, codebooks[i],
                                  p['conv_b'], p['gbn_g'], p['gbn_b'],
                                  p['gate_W'][:, 0], p['gate_b'])
        loss_latents = loss_latents + loss
    return X, loss_latents, perp
